# baseline (speedup 1.0000x reference)
import functools

import jax
import jax.numpy as jnp
from jax import lax
from jax.experimental import pallas as pl
from jax.experimental.pallas import tpu as pltpu
from jax.experimental.pallas import tpu_sc as plsc

NC = 2
NS = 16
NW = NC * NS
DIM = 32
BATCH = 16384
VOCAB = 1000000
B_PER_W = BATCH // NW
CHUNK = 128
NCHUNK = B_PER_W // CHUNK
BLK = 40960
GRID = (VOCAB + BLK - 1) // BLK

P1CH = 1024
P1NCH = 976
P1K = 31
P1REM = 999424
P1TAIL = 999936


def _p1e_body(ent_ref, we_ref, ys_ref, yo_ref):
    eo = jnp.dot(we_ref[...], ent_ref[...], preferred_element_type=jnp.float32)
    ys_ref[...] = eo[0]
    yo_ref[...] = eo[1]


def _p1p_body(pred3_hbm, wp_hbm, tail_hbm, yp_hbm, buf, acc, wv, tl, sem0, sem1):
    wid = lax.axis_index("s") * NC + lax.axis_index("c")
    pltpu.sync_copy(wp_hbm, wv)
    wlo = wv[pl.ds(0, 16)]
    whi = wv[pl.ds(16, 16)]
    wsc = [wlo[j] for j in range(16)] + [whi[j] for j in range(16)]
    sems = (sem0, sem1)

    def fire(p, off, width):
        for t in range(4):
            pltpu.async_copy(
                pred3_hbm.at[t, :, pl.ds(off, width)],
                buf.at[p, t, :, pl.ds(0, width)], sems[p])

    def drain(p, off, width):
        for t in range(4):
            pltpu.make_async_copy(
                pred3_hbm.at[t, :, pl.ds(off, width)],
                buf.at[p, t, :, pl.ds(0, width)], sems[p]).wait()

    def compute(p, ngroups, width):
        def grp(g, carry):
            base = g * 16
            a = jnp.zeros((16,), jnp.float32)
            for t in range(4):
                for dd in range(8):
                    v = buf[p, t, dd, pl.ds(base, 16)]
                    a = a + v * wsc[8 * t + dd]
            acc[pl.ds(base, 16)] = a
            return carry
        lax.fori_loop(0, ngroups, grp, 0)

    def chunk_off(k):
        return jnp.minimum(wid + 32 * k, P1NCH - 1) * P1CH

    def do_step(p, off):
        drain(p, off, P1CH)
        compute(p, P1CH // 16, P1CH)
        pltpu.sync_copy(acc.at[pl.ds(0, P1CH)], yp_hbm.at[pl.ds(off, P1CH)])

    fire(0, chunk_off(0), P1CH)

    def pair(k2, carry):
        k = 2 * k2
        fire(1, chunk_off(k + 1), P1CH)
        do_step(0, chunk_off(k))
        fire(0, chunk_off(k + 2), P1CH)
        do_step(1, chunk_off(k + 1))
        return carry

    lax.fori_loop(0, (P1K - 1) // 2, pair, 0)
    do_step(0, chunk_off(P1K - 1))

    @pl.when(wid == 0)
    def _():
        fire(0, P1REM, 512)
        drain(0, P1REM, 512)
        compute(0, 512 // 16, 512)
        pltpu.sync_copy(acc.at[pl.ds(0, 512)], yp_hbm.at[pl.ds(P1REM, 512)])

    @pl.when(wid == 1)
    def _():
        pltpu.sync_copy(tail_hbm, tl)
        for g in range(4):
            a = jnp.zeros((16,), jnp.float32)
            for d in range(DIM):
                a = a + tl[pl.ds(d * 64 + g * 16, 16)] * wsc[d]
            acc[pl.ds(g * 16, 16)] = a
        pltpu.sync_copy(acc.at[pl.ds(0, 64)], yp_hbm.at[pl.ds(P1TAIL, 64)])


def _sc_body(ids_hbm, ys_hbm, yp_hbm, yo_hbm, wb_hbm, out_hbm,
             sidx, pidx, oidx, gs, gp, go, scores, wv, sem):
    wid = lax.axis_index("s") * NC + lax.axis_index("c")
    base = wid * B_PER_W

    pltpu.sync_copy(ids_hbm.at[0, wid], sidx)
    pltpu.sync_copy(ids_hbm.at[1, wid], pidx)
    pltpu.sync_copy(ids_hbm.at[2, wid], oidx)
    pltpu.sync_copy(wb_hbm, wv)

    descs = []
    for k in range(NCHUNK):
        dst = pl.ds(k * CHUNK, CHUNK)
        descs.append(pltpu.async_copy(ys_hbm.at[sidx.at[k]], gs.at[dst], sem))
        descs.append(pltpu.async_copy(yp_hbm.at[pidx.at[k]], gp.at[dst], sem))
        descs.append(pltpu.async_copy(yo_hbm.at[oidx.at[k]], go.at[dst], sem))
    for d in descs:
        d.wait()

    bias = wv[pl.ds(0, 16)][0]
    for v in range(B_PER_W // 16):
        sl = pl.ds(v * 16, 16)
        scores[sl] = gs[sl] + gp[sl] + go[sl] + bias

    pltpu.sync_copy(scores, out_hbm.at[pl.ds(base, B_PER_W)])


@jax.jit
def _triple_score(ids_r, ent_t, pred3, tail_lin, we, wpv, wb):
    mesh = plsc.VectorSubcoreMesh(core_axis_name="c", subcore_axis_name="s")

    yp = functools.partial(
        pl.kernel,
        out_type=jax.ShapeDtypeStruct((VOCAB,), jnp.float32),
        mesh=mesh,
        scratch_types=[
            pltpu.VMEM((2, 4, 8, P1CH), jnp.float32),
            pltpu.VMEM((P1CH,), jnp.float32),
            pltpu.VMEM((DIM,), jnp.float32),
            pltpu.VMEM((2048,), jnp.float32),
            pltpu.SemaphoreType.DMA,
            pltpu.SemaphoreType.DMA,
        ],
        compiler_params=pltpu.CompilerParams(use_tc_tiling_on_sc=True),
    )(_p1p_body)(pred3, wpv, tail_lin)

    ys, yo = pl.pallas_call(
        _p1e_body,
        grid=(GRID,),
        in_specs=[
            pl.BlockSpec((DIM, BLK), lambda i: (0, i)),
            pl.BlockSpec((2, DIM), lambda i: (0, 0)),
        ],
        out_specs=[
            pl.BlockSpec((BLK,), lambda i: (i,)),
            pl.BlockSpec((BLK,), lambda i: (i,)),
        ],
        out_shape=[
            jax.ShapeDtypeStruct((VOCAB,), jnp.float32),
            jax.ShapeDtypeStruct((VOCAB,), jnp.float32),
        ],
    )(ent_t, we)

    f = functools.partial(
        pl.kernel,
        out_type=jax.ShapeDtypeStruct((BATCH,), jnp.float32),
        mesh=mesh,
        scratch_types=[
            pltpu.VMEM((NCHUNK, CHUNK), jnp.int32),
            pltpu.VMEM((NCHUNK, CHUNK), jnp.int32),
            pltpu.VMEM((NCHUNK, CHUNK), jnp.int32),
            pltpu.VMEM((B_PER_W,), jnp.float32),
            pltpu.VMEM((B_PER_W,), jnp.float32),
            pltpu.VMEM((B_PER_W,), jnp.float32),
            pltpu.VMEM((B_PER_W,), jnp.float32),
            pltpu.VMEM((16,), jnp.float32),
            pltpu.SemaphoreType.DMA,
        ],
        compiler_params=pltpu.CompilerParams(
            needs_layout_passes=False, use_tc_tiling_on_sc=False),
    )(_sc_body)
    return f(ids_r, ys, yp, yo, wb)


def kernel(triple_ids, entity_emb, pred_emb, W, b):
    if triple_ids.ndim == 1:
        triple_ids = triple_ids[None, :]
    ids_r = triple_ids.T.astype(jnp.int32).reshape(3, NW, NCHUNK, CHUNK)
    w3 = W.reshape(3, DIM)
    we = jnp.stack([w3[0], w3[2]])
    wpv = w3[1]
    wb = jnp.broadcast_to(b.reshape(1), (16,)).astype(jnp.float32)
    pred3 = pred_emb.T.reshape(4, 8, VOCAB)
    tail_lin = pred_emb.T[:, P1TAIL:].reshape(-1)
    return _triple_score(ids_r, entity_emb.T, pred3, tail_lin, we, wpv, wb)
